# Initial kernel scaffold; baseline (speedup 1.0000x reference)
import jax
import jax.numpy as jnp
from jax.experimental import pallas as pl


def kernel(x, edge_index, W1, b1, W2, b2):
    raise NotImplementedError("write your pallas kernel here")


# speedup vs baseline: 2.0996x; 2.0996x over previous
import functools

import jax
import jax.numpy as jnp
from jax import lax
from jax.experimental import pallas as pl
from jax.experimental.pallas import tpu as pltpu
from jax.experimental.pallas import tpu_sc as plsc

_N = 10000
_C = 256
_K = 16
_OUT = 256

_NC = 2
_NS = 16
_NW = _NC * _NS
_NODES_W = 320
_NP = _NW * _NODES_W

_LANES = 16
_CB = _C // _LANES
_GRP = 64
_NGRP = _NODES_W // _GRP


def _ab_body(x_ref, w_ref, ab_ref):
    ab_ref[...] = jnp.dot(x_ref[...], w_ref[...],
                          preferred_element_type=jnp.float32)


def _tc_ab(X, wcat):
    blk = 2000
    return pl.pallas_call(
        _ab_body,
        grid=(_N // blk,),
        in_specs=[pl.BlockSpec((blk, _C), lambda i: (i, 0)),
                  pl.BlockSpec((_C, 2), lambda i: (0, 0))],
        out_specs=pl.BlockSpec((blk, 2), lambda i: (i, 0)),
        out_shape=jax.ShapeDtypeStruct((_N, 2), jnp.float32),
    )(X, wcat)


@functools.cache
def _make_sc_agg():
    mesh = plsc.VectorSubcoreMesh(core_axis_name="c", subcore_axis_name="s",
                                  num_cores=_NC, num_subcores=_NS)
    return functools.partial(
        pl.kernel,
        out_type=jax.ShapeDtypeStruct((_NP, _C), jnp.float32),
        mesh=mesh,
        compiler_params=pltpu.CompilerParams(needs_layout_passes=False),
        scratch_types=[
            pltpu.VMEM((_N,), jnp.float32),
            pltpu.VMEM((_N,), jnp.float32),
            pltpu.VMEM((_NODES_W, _K), jnp.int32),
            pltpu.VMEM((_NODES_W, _K), jnp.int32),
            pltpu.VMEM((_K, _C), jnp.float32),
            pltpu.VMEM((_GRP, _C), jnp.float32),
            pltpu.SemaphoreType.DMA,
        ],
    )(_sc_agg_body)


def _sc_agg_body(a_hbm, b_hbm, idxi_hbm, idxj_hbm, x_hbm, agg_hbm,
                 a_v, b_v, idxi_v, idxj_v, rows_v, agg_v, sem):
    wid = lax.axis_index("s") * _NC + lax.axis_index("c")
    nb = wid * _NODES_W
    pltpu.sync_copy(a_hbm, a_v)
    pltpu.sync_copy(b_hbm, b_v)
    pltpu.sync_copy(idxi_hbm.at[pl.ds(nb, _NODES_W)], idxi_v)
    pltpu.sync_copy(idxj_hbm.at[pl.ds(nb, _NODES_W)], idxj_v)

    def group_body(g, carry):
        def node_body(t, carry2):
            n = g * _GRP + t
            ii = idxi_v[n]
            jj = idxj_v[n]
            ai = plsc.load_gather(a_v, [ii])
            bj = plsc.load_gather(b_v, [jj])
            s = jnp.maximum(ai + bj, 0.0)
            m = jnp.max(s)
            e = jnp.exp(s - m)
            p = e / jnp.sum(e)
            pltpu.async_copy(x_hbm.at[idxj_v.at[n]], rows_v, sem).wait()
            for cb in range(_CB):
                sl = pl.ds(cb * _LANES, _LANES)
                acc = p[0] * rows_v[0, sl]
                for k in range(1, _K):
                    acc = acc + p[k] * rows_v[k, sl]
                agg_v[t, sl] = acc
            return carry2

        lax.fori_loop(0, _GRP, node_body, 0)
        pltpu.sync_copy(agg_v, agg_hbm.at[pl.ds(nb + g * _GRP, _GRP)])
        return carry

    lax.fori_loop(0, _NGRP, group_body, 0)


def _out_body(agg_ref, w2_ref, b2_ref, o_ref):
    o_ref[...] = jnp.maximum(
        jnp.dot(agg_ref[...], w2_ref[...],
                preferred_element_type=jnp.float32) + b2_ref[...],
        0.0)


def _tc_out(agg, W2, b2):
    blk = 1024
    return pl.pallas_call(
        _out_body,
        grid=(_NP // blk,),
        in_specs=[pl.BlockSpec((blk, _C), lambda i: (i, 0)),
                  pl.BlockSpec((_C, _OUT), lambda i: (0, 0)),
                  pl.BlockSpec((1, _OUT), lambda i: (0, 0))],
        out_specs=pl.BlockSpec((blk, _OUT), lambda i: (i, 0)),
        out_shape=jax.ShapeDtypeStruct((_NP, _OUT), jnp.float32),
    )(agg, W2, b2)


def kernel(x, edge_index, W1, b1, W2, b2):
    X = x.reshape(_N, _C)
    idx_dst = edge_index[1, 0].astype(jnp.int32)
    idx_src = edge_index[0, 0].astype(jnp.int32)
    pad = _NP - _N
    idx_dst = jnp.pad(idx_dst, ((0, pad), (0, 0)))
    idx_src = jnp.pad(idx_src, ((0, pad), (0, 0)))
    wcat = jnp.concatenate([W1[:_C], W1[_C:]], axis=1)
    ab = _tc_ab(X, wcat)
    a = ab[:, 0] + b1[0]
    b = ab[:, 1]
    agg = _make_sc_agg()(a, b, idx_dst, idx_src, X)
    out = _tc_out(agg, W2, b2.reshape(1, _OUT))
    return out[:_N].reshape(1, _N, _OUT)


# speedup vs baseline: 3.4942x; 1.6642x over previous
import functools

import jax
import jax.numpy as jnp
from jax import lax
from jax.experimental import pallas as pl
from jax.experimental.pallas import tpu as pltpu
from jax.experimental.pallas import tpu_sc as plsc

_N = 10000
_C = 256
_K = 16
_OUT = 256

_NC = 2
_NS = 16
_NW = _NC * _NS
_NODES_W = 320
_NP = _NW * _NODES_W

_LANES = 16
_CB = _C // _LANES
_GRP = 32
_NGRP = _NODES_W // _GRP
_RING = 4


def _ab_body(x_ref, w_ref, ab_ref):
    ab_ref[...] = jnp.dot(x_ref[...], w_ref[...],
                          preferred_element_type=jnp.float32)


def _tc_ab(X, wcat):
    blk = 2000
    return pl.pallas_call(
        _ab_body,
        grid=(_N // blk,),
        in_specs=[pl.BlockSpec((blk, _C), lambda i: (i, 0)),
                  pl.BlockSpec((_C, 2), lambda i: (0, 0))],
        out_specs=pl.BlockSpec((blk, 2), lambda i: (i, 0)),
        out_shape=jax.ShapeDtypeStruct((_N, 2), jnp.float32),
    )(X, wcat)


@functools.cache
def _make_sc_agg():
    mesh = plsc.VectorSubcoreMesh(core_axis_name="c", subcore_axis_name="s",
                                  num_cores=_NC, num_subcores=_NS)
    return functools.partial(
        pl.kernel,
        out_type=jax.ShapeDtypeStruct((_NP, _C), jnp.float32),
        mesh=mesh,
        compiler_params=pltpu.CompilerParams(needs_layout_passes=False),
        scratch_types=[
            pltpu.VMEM((_N,), jnp.float32),
            pltpu.VMEM((_N,), jnp.float32),
            pltpu.VMEM((_NODES_W, _K), jnp.int32),
            pltpu.VMEM((_NODES_W, _K), jnp.int32),
            pltpu.VMEM((_RING, _K, _C), jnp.float32),
            pltpu.VMEM((_GRP, _C), jnp.float32),
            pltpu.SemaphoreType.DMA((_RING,)),
        ],
    )(_sc_agg_body)


def _sc_agg_body(a_hbm, b_hbm, idxi_hbm, idxj_hbm, x_hbm, agg_hbm,
                 a_v, b_v, idxi_v, idxj_v, rows_v, agg_v, sem):
    wid = lax.axis_index("s") * _NC + lax.axis_index("c")
    nb = wid * _NODES_W
    pltpu.sync_copy(a_hbm, a_v)
    pltpu.sync_copy(b_hbm, b_v)
    pltpu.sync_copy(idxi_hbm.at[pl.ds(nb, _NODES_W)], idxi_v)
    pltpu.sync_copy(idxj_hbm.at[pl.ds(nb, _NODES_W)], idxj_v)

    def issue(n, slot):
        pltpu.async_copy(x_hbm.at[idxj_v.at[n]],
                         rows_v.at[slot], sem.at[slot])

    for n0 in range(_RING):
        issue(n0, n0)

    def node_body(n, carry):
        slot = lax.rem(n, _RING)
        pltpu.make_async_copy(x_hbm.at[idxj_v.at[n]],
                              rows_v.at[slot], sem.at[slot]).wait()
        ii = idxi_v[n]
        jj = idxj_v[n]
        ai = plsc.load_gather(a_v, [ii])
        bj = plsc.load_gather(b_v, [jj])
        sc = jnp.maximum(ai + bj, 0.0)
        m = jnp.max(sc)
        e = jnp.exp(sc - m)
        p = e / jnp.sum(e)
        t = lax.rem(n, _GRP)
        for cb in range(_CB):
            sl = pl.ds(cb * _LANES, _LANES)
            acc = p[0] * rows_v[slot, 0, sl]
            for k in range(1, _K):
                acc = acc + p[k] * rows_v[slot, k, sl]
            agg_v[t, sl] = acc

        @pl.when(n + _RING < _NODES_W)
        def _():
            issue(n + _RING, slot)

        @pl.when(t == _GRP - 1)
        def _():
            g = n // _GRP
            pltpu.sync_copy(agg_v, agg_hbm.at[pl.ds(nb + g * _GRP, _GRP)])

        return carry

    lax.fori_loop(0, _NODES_W, node_body, 0)


def _out_body(agg_ref, w2_ref, b2_ref, o_ref):
    o_ref[...] = jnp.maximum(
        jnp.dot(agg_ref[...], w2_ref[...],
                preferred_element_type=jnp.float32) + b2_ref[...],
        0.0)


def _tc_out(agg, W2, b2):
    blk = 1024
    return pl.pallas_call(
        _out_body,
        grid=(_NP // blk,),
        in_specs=[pl.BlockSpec((blk, _C), lambda i: (i, 0)),
                  pl.BlockSpec((_C, _OUT), lambda i: (0, 0)),
                  pl.BlockSpec((1, _OUT), lambda i: (0, 0))],
        out_specs=pl.BlockSpec((blk, _OUT), lambda i: (i, 0)),
        out_shape=jax.ShapeDtypeStruct((_NP, _OUT), jnp.float32),
    )(agg, W2, b2)


def kernel(x, edge_index, W1, b1, W2, b2):
    X = x.reshape(_N, _C)
    idx_dst = edge_index[1, 0].astype(jnp.int32)
    idx_src = edge_index[0, 0].astype(jnp.int32)
    pad = _NP - _N
    idx_dst = jnp.pad(idx_dst, ((0, pad), (0, 0)))
    idx_src = jnp.pad(idx_src, ((0, pad), (0, 0)))
    wcat = jnp.concatenate([W1[:_C], W1[_C:]], axis=1)
    ab = _tc_ab(X, wcat)
    a = ab[:, 0] + b1[0]
    b = ab[:, 1]
    agg = _make_sc_agg()(a, b, idx_dst, idx_src, X)
    out = _tc_out(agg, W2, b2.reshape(1, _OUT))
    return out[:_N].reshape(1, _N, _OUT)


# speedup vs baseline: 3.7137x; 1.0628x over previous
import functools

import jax
import jax.numpy as jnp
from jax import lax
from jax.experimental import pallas as pl
from jax.experimental.pallas import tpu as pltpu
from jax.experimental.pallas import tpu_sc as plsc

_N = 10000
_C = 256
_K = 16
_OUT = 256

_NC = 2
_NS = 16
_NW = _NC * _NS
_NODES_W = 320
_NP = _NW * _NODES_W

_LANES = 16
_CB = _C // _LANES
_GRP = 32
_NGRP = _NODES_W // _GRP
_RING = 4


def _ab_body(x_ref, w_ref, ab_ref):
    ab_ref[...] = jnp.dot(x_ref[...], w_ref[...],
                          preferred_element_type=jnp.float32)


def _tc_ab(X, wcat):
    blk = 2000
    return pl.pallas_call(
        _ab_body,
        grid=(_N // blk,),
        in_specs=[pl.BlockSpec((blk, _C), lambda i: (i, 0)),
                  pl.BlockSpec((_C, 2), lambda i: (0, 0))],
        out_specs=pl.BlockSpec((blk, 2), lambda i: (i, 0)),
        out_shape=jax.ShapeDtypeStruct((_N, 2), jnp.float32),
    )(X, wcat)


@functools.cache
def _make_sc_agg():
    mesh = plsc.VectorSubcoreMesh(core_axis_name="c", subcore_axis_name="s",
                                  num_cores=_NC, num_subcores=_NS)
    return functools.partial(
        pl.kernel,
        out_type=jax.ShapeDtypeStruct((_NP, _C), jnp.float32),
        mesh=mesh,
        compiler_params=pltpu.CompilerParams(needs_layout_passes=False),
        scratch_types=[
            pltpu.VMEM((_N,), jnp.float32),
            pltpu.VMEM((_N,), jnp.float32),
            pltpu.VMEM((_NODES_W, _K), jnp.int32),
            pltpu.VMEM((_NODES_W, _K), jnp.int32),
            pltpu.VMEM((_RING, _K, _C), jnp.float32),
            pltpu.VMEM((_GRP, _C), jnp.float32),
            pltpu.SemaphoreType.DMA((_RING,)),
        ],
    )(_sc_agg_body)


def _sc_agg_body(a_hbm, b_hbm, idxi_hbm, idxj_hbm, x_hbm, agg_hbm,
                 a_v, b_v, idxi_v, idxj_v, rows_v, agg_v, sem):
    wid = lax.axis_index("s") * _NC + lax.axis_index("c")
    nb = wid * _NODES_W
    pltpu.sync_copy(a_hbm, a_v)
    pltpu.sync_copy(b_hbm, b_v)
    pltpu.sync_copy(idxi_hbm.at[pl.ds(nb, _NODES_W)], idxi_v)
    pltpu.sync_copy(idxj_hbm.at[pl.ds(nb, _NODES_W)], idxj_v)

    def issue(n, slot):
        pltpu.async_copy(x_hbm.at[idxj_v.at[n]],
                         rows_v.at[slot], sem.at[slot])

    for n0 in range(_RING):
        issue(n0, n0)

    def node_body(n, carry):
        slot = lax.rem(n, _RING)
        pltpu.make_async_copy(x_hbm.at[idxj_v.at[n]],
                              rows_v.at[slot], sem.at[slot]).wait()
        ii = idxi_v[n]
        jj = idxj_v[n]
        ai = plsc.load_gather(a_v, [ii])
        bj = plsc.load_gather(b_v, [jj])
        sc = jnp.maximum(ai + bj, 0.0)
        m = jnp.max(sc)
        e = jnp.exp(sc - m)
        p = e / jnp.sum(e)
        t = lax.rem(n, _GRP)
        sls = [pl.ds(cb * _LANES, _LANES) for cb in range(_CB)]
        p0 = p[0]
        accs = [p0 * rows_v[slot, 0, sl] for sl in sls]
        for k in range(1, _K):
            pk = p[k]
            accs = [accs[cb] + pk * rows_v[slot, k, sls[cb]]
                    for cb in range(_CB)]
        for cb in range(_CB):
            agg_v[t, sls[cb]] = accs[cb]

        @pl.when(n + _RING < _NODES_W)
        def _():
            issue(n + _RING, slot)

        @pl.when(t == _GRP - 1)
        def _():
            g = n // _GRP
            pltpu.sync_copy(agg_v, agg_hbm.at[pl.ds(nb + g * _GRP, _GRP)])

        return carry

    lax.fori_loop(0, _NODES_W, node_body, 0)


def _out_body(agg_ref, w2_ref, b2_ref, o_ref):
    o_ref[...] = jnp.maximum(
        jnp.dot(agg_ref[...], w2_ref[...],
                preferred_element_type=jnp.float32) + b2_ref[...],
        0.0)


def _tc_out(agg, W2, b2):
    blk = 1024
    return pl.pallas_call(
        _out_body,
        grid=(_NP // blk,),
        in_specs=[pl.BlockSpec((blk, _C), lambda i: (i, 0)),
                  pl.BlockSpec((_C, _OUT), lambda i: (0, 0)),
                  pl.BlockSpec((1, _OUT), lambda i: (0, 0))],
        out_specs=pl.BlockSpec((blk, _OUT), lambda i: (i, 0)),
        out_shape=jax.ShapeDtypeStruct((_NP, _OUT), jnp.float32),
    )(agg, W2, b2)


def kernel(x, edge_index, W1, b1, W2, b2):
    X = x.reshape(_N, _C)
    idx_dst = edge_index[1, 0].astype(jnp.int32)
    idx_src = edge_index[0, 0].astype(jnp.int32)
    pad = _NP - _N
    idx_dst = jnp.pad(idx_dst, ((0, pad), (0, 0)))
    idx_src = jnp.pad(idx_src, ((0, pad), (0, 0)))
    wcat = jnp.concatenate([W1[:_C], W1[_C:]], axis=1)
    ab = _tc_ab(X, wcat)
    a = ab[:, 0] + b1[0]
    b = ab[:, 1]
    agg = _make_sc_agg()(a, b, idx_dst, idx_src, X)
    out = _tc_out(agg, W2, b2.reshape(1, _OUT))
    return out[:_N].reshape(1, _N, _OUT)
